# baseline (speedup 1.0000x reference)
import functools

import jax
import jax.numpy as jnp
from jax import lax
from jax.experimental import pallas as pl
from jax.experimental.pallas import tpu as pltpu
from jax.experimental.pallas import tpu_sc as plsc

_HIDDEN = 128
_TABLE_ROWS = 1000
_IDX_CHUNK = 128


def _sc_gather(table, idx3, n_rows):
    nw, n_ch, ch = idx3.shape
    rows_per_w = n_ch * ch
    mesh = plsc.VectorSubcoreMesh(core_axis_name="c", subcore_axis_name="s")

    @functools.partial(
        pl.kernel,
        mesh=mesh,
        out_type=jax.ShapeDtypeStruct((n_rows, _HIDDEN), jnp.float32),
        scratch_types=[
            pltpu.VMEM((n_ch, ch), jnp.int32),
            pltpu.VMEM((rows_per_w, _HIDDEN), jnp.float32),
            pltpu.SemaphoreType.DMA,
        ],
    )
    def gather_kernel(table_hbm, idx_hbm, out_hbm, idx_v, rows_v, sem):
        num_cores = lax.axis_size("c")
        wid = lax.axis_index("s") * num_cores + lax.axis_index("c")
        base = wid * rows_per_w
        pltpu.sync_copy(idx_hbm.at[wid], idx_v)
        copies = [
            pltpu.async_copy(
                table_hbm.at[idx_v.at[c]],
                rows_v.at[pl.ds(c * ch, ch)],
                sem,
            )
            for c in range(n_ch)
        ]
        for cp in copies:
            cp.wait()
        pltpu.sync_copy(rows_v, out_hbm.at[pl.ds(base, rows_per_w)])

    return gather_kernel(table, idx3)


def _add_body(f_ref, p_ref, o_ref):
    pos = p_ref[...]
    o_ref[...] = f_ref[...] + pos[None, :, :]


def _tc_add(features, pos, block_rows):
    n, t, d = features.shape
    ft = jnp.transpose(features, (1, 0, 2))
    grid = (n // block_rows, t)
    out_t = pl.pallas_call(
        _add_body,
        grid=grid,
        in_specs=[
            pl.BlockSpec((1, block_rows, d), lambda j, i: (i, j, 0)),
            pl.BlockSpec((block_rows, d), lambda j, i: (j, 0)),
        ],
        out_specs=pl.BlockSpec((1, block_rows, d), lambda j, i: (i, j, 0)),
        out_shape=jax.ShapeDtypeStruct((t, n, d), features.dtype),
    )(ft, pos)
    return jnp.transpose(out_t, (1, 0, 2))


def kernel(features, temporal_embedding):
    n = features.shape[0]
    temporal_pos = jnp.linspace(0.0, 1.0, n, dtype=features.dtype)
    idx = jnp.clip(temporal_pos * _TABLE_ROWS, 0, _TABLE_ROWS - 1).astype(jnp.int32)

    info = plsc.get_sparse_core_info()
    nw = info.num_cores * info.num_subcores
    idx3 = idx.reshape(nw, -1, _IDX_CHUNK)

    pos = _sc_gather(temporal_embedding, idx3, n)
    return _tc_add(features, pos, block_rows=4096)


# speedup vs baseline: 1.0985x; 1.0985x over previous
import functools

import jax
import jax.numpy as jnp
from jax import lax
from jax.experimental import pallas as pl
from jax.experimental.pallas import tpu as pltpu
from jax.experimental.pallas import tpu_sc as plsc

_HIDDEN = 128
_TABLE_ROWS = 1000
_IDX_CHUNK = 128


def _sc_gather(table, idx3, n_rows):
    nw, n_ch, ch = idx3.shape
    rows_per_w = n_ch * ch
    mesh = plsc.VectorSubcoreMesh(core_axis_name="c", subcore_axis_name="s")

    @functools.partial(
        pl.kernel,
        mesh=mesh,
        out_type=jax.ShapeDtypeStruct((n_rows, _HIDDEN), jnp.float32),
        scratch_types=[
            pltpu.VMEM((n_ch, ch), jnp.int32),
            pltpu.VMEM((rows_per_w, _HIDDEN), jnp.float32),
            pltpu.SemaphoreType.DMA,
        ],
    )
    def gather_kernel(table_hbm, idx_hbm, out_hbm, idx_v, rows_v, sem):
        num_cores = lax.axis_size("c")
        wid = lax.axis_index("s") * num_cores + lax.axis_index("c")
        base = wid * rows_per_w
        pltpu.sync_copy(idx_hbm.at[wid], idx_v)
        copies = [
            pltpu.async_copy(
                table_hbm.at[idx_v.at[c]],
                rows_v.at[pl.ds(c * ch, ch)],
                sem,
            )
            for c in range(n_ch)
        ]
        for cp in copies:
            cp.wait()
        pltpu.sync_copy(rows_v, out_hbm.at[pl.ds(base, rows_per_w)])

    return gather_kernel(table, idx3)


def _add_body(f_ref, p_ref, o_ref):
    pos = p_ref[...]
    o_ref[...] = f_ref[...] + pos[None, :, :]


def _tc_add(features, pos, block_rows):
    n, t, d = features.shape
    ft = jnp.transpose(features, (1, 0, 2))
    grid = (n // block_rows, t)
    out_t = pl.pallas_call(
        _add_body,
        grid=grid,
        in_specs=[
            pl.BlockSpec((1, block_rows, d), lambda j, i: (i, j, 0)),
            pl.BlockSpec((block_rows, d), lambda j, i: (j, 0)),
        ],
        out_specs=pl.BlockSpec((1, block_rows, d), lambda j, i: (i, j, 0)),
        out_shape=jax.ShapeDtypeStruct((t, n, d), features.dtype),
    )(ft, pos)
    return jnp.transpose(out_t, (1, 0, 2))


def kernel(features, temporal_embedding):
    n = features.shape[0]
    temporal_pos = jnp.linspace(0.0, 1.0, n, dtype=features.dtype)
    idx = jnp.clip(temporal_pos * _TABLE_ROWS, 0, _TABLE_ROWS - 1).astype(jnp.int32)

    info = plsc.get_sparse_core_info()
    nw = info.num_cores * info.num_subcores
    idx3 = idx.reshape(nw, -1, _IDX_CHUNK)

    pos = _sc_gather(temporal_embedding, idx3, n)
    return _tc_add(features, pos, block_rows=16384)
